# Initial kernel scaffold; baseline (speedup 1.0000x reference)
import jax
import jax.numpy as jnp
from jax import lax
from jax.experimental import pallas as pl
from jax.experimental.pallas import tpu as pltpu
from jax.experimental.pallas import tpu_sc as plsc

B = 4096
N_NUM = 13
N_CAT = 26
SEQ_LEN = 50
VOCAB = 100000
D = 32
NCOLS = N_NUM + N_CAT + SEQ_LEN
NSLOT = N_NUM + N_CAT + 1

NC, NS = 2, 16
NW = NC * NS
RPW = B // NW
C = 32
NCHUNK = RPW // C

CAT_ROWS = N_CAT * C
SEQ_ROWS = SEQ_LEN * C
NP_ROWS = (N_NUM + 1) * C
NP_W = 112


def _body(x_hbm, w_hbm, cat_hbm, seq_hbm, out_hbm,
          xv, seq_v, cat_v, np_v, gidx_seq, gidx_cat,
          sidx_cat, sidx_np, rcp_v, wv,
          sem_seq, sem_cat, sem_out):
  wid = lax.axis_index("s") * NC + lax.axis_index("c")
  lane = lax.iota(jnp.int32, 16)

  pltpu.sync_copy(w_hbm, wv)

  for ci in range(NCHUNK):
    gbase = wid * RPW + ci * C

    pltpu.sync_copy(x_hbm.at[pl.ds(gbase, C)], xv)

    for j in range(N_CAT):
      for k in range(C // 16):
        rows = k * 16 + lane
        v = plsc.load_gather(xv, [rows, jnp.full((16,), N_NUM + j, jnp.int32)])
        gidx_cat[pl.ds(j * C + k * 16, 16)] = v + j * VOCAB
    cat_descs = []
    for r in range(6):
      cat_descs.append(pltpu.async_copy(
          cat_hbm.at[gidx_cat.at[pl.ds(r * 128, 128)]],
          cat_v.at[pl.ds(r * 128, 128)], sem_cat))
    cat_descs.append(pltpu.async_copy(
        cat_hbm.at[gidx_cat.at[pl.ds(768, 64)]],
        cat_v.at[pl.ds(768, 64)], sem_cat))

    cnt = [jnp.zeros((16,), jnp.float32) for _ in range(C // 16)]
    for t in range(SEQ_LEN):
      for k in range(C // 16):
        rows = k * 16 + lane
        v = plsc.load_gather(
            xv, [rows, jnp.full((16,), N_NUM + N_CAT + t, jnp.int32)])
        gidx_seq[pl.ds(t * C + k * 16, 16)] = v
        cnt[k] = cnt[k] + jnp.where(v != 0, 1.0, 0.0)
    for k in range(C // 16):
      rcp_v[pl.ds(k * 16, 16)] = 1.0 / jnp.maximum(cnt[k], 1e-12)
    seq_descs = []
    for r in range(12):
      seq_descs.append(pltpu.async_copy(
          seq_hbm.at[gidx_seq.at[pl.ds(r * 128, 128)]],
          seq_v.at[pl.ds(r * 128, 128)], sem_seq))
    seq_descs.append(pltpu.async_copy(
        seq_hbm.at[gidx_seq.at[pl.ds(1536, 64)]],
        seq_v.at[pl.ds(1536, 64)], sem_seq))

    for j in range(N_CAT):
      for k in range(C // 16):
        orow = (gbase + k * 16 + lane) * NSLOT + N_NUM + j
        sidx_cat[j // 2, pl.ds((j % 2) * C + k * 16, 16)] = orow
    for i in range(N_NUM + 1):
      slot = i if i < N_NUM else NSLOT - 1
      for k in range(C // 16):
        bb = k * 16 + lane
        flat = bb * (N_NUM + 1) + i
        orow = (gbase + bb) * NSLOT + slot
        plsc.store_scatter(sidx_np, [flat // NP_W, flat % NP_W], orow)

    for dsc in seq_descs:
      dsc.wait()

    def row_body(b, carry):
      base = b * (N_NUM + 1)
      for i in range(N_NUM):
        xi = plsc.load_gather(
            xv, [jnp.full((16,), b, jnp.int32),
                 jnp.full((16,), i, jnp.int32)]).astype(jnp.float32)
        np_v[base + i, pl.ds(0, 16)] = xi * wv[i, pl.ds(0, 16)]
        np_v[base + i, pl.ds(16, 16)] = xi * wv[i, pl.ds(16, 16)]
      acc0 = jnp.zeros((16,), jnp.float32)
      acc1 = jnp.zeros((16,), jnp.float32)
      for t in range(SEQ_LEN):
        acc0 = acc0 + seq_v[t * C + b, pl.ds(0, 16)]
        acc1 = acc1 + seq_v[t * C + b, pl.ds(16, 16)]
      rcp = plsc.load_gather(rcp_v, [jnp.full((16,), b, jnp.int32)])
      np_v[base + N_NUM, pl.ds(0, 16)] = acc0 * rcp
      np_v[base + N_NUM, pl.ds(16, 16)] = acc1 * rcp
      return carry

    lax.fori_loop(0, C, row_body, 0)

    out_descs = []
    for r in range(NP_ROWS // NP_W):
      out_descs.append(pltpu.async_copy(
          np_v.at[pl.ds(r * NP_W, NP_W)],
          out_hbm.at[sidx_np.at[r]], sem_out))

    for dsc in cat_descs:
      dsc.wait()
    for r in range(N_CAT // 2):
      out_descs.append(pltpu.async_copy(
          cat_v.at[pl.ds(r * 2 * C, 2 * C)],
          out_hbm.at[sidx_cat.at[r]], sem_out))

    for dsc in out_descs:
      dsc.wait()


_sc_call = pl.kernel(
    _body,
    out_type=jax.ShapeDtypeStruct((B * NSLOT, D), jnp.float32),
    mesh=plsc.VectorSubcoreMesh(core_axis_name="c", subcore_axis_name="s"),
    scratch_types=[
        pltpu.VMEM((C, NCOLS), jnp.int32),
        pltpu.VMEM((SEQ_ROWS, D), jnp.float32),
        pltpu.VMEM((CAT_ROWS, D), jnp.float32),
        pltpu.VMEM((NP_ROWS, D), jnp.float32),
        pltpu.VMEM((SEQ_ROWS,), jnp.int32),
        pltpu.VMEM((CAT_ROWS,), jnp.int32),
        pltpu.VMEM((N_CAT // 2, 2 * C), jnp.int32),
        pltpu.VMEM((NP_ROWS // NP_W, NP_W), jnp.int32),
        pltpu.VMEM((C,), jnp.float32),
        pltpu.VMEM((N_NUM, D), jnp.float32),
        pltpu.SemaphoreType.DMA,
        pltpu.SemaphoreType.DMA,
        pltpu.SemaphoreType.DMA,
    ],
)


@jax.jit
def kernel(X, W_num, cat_tables, seq_table):
  cat_flat = cat_tables.reshape(N_CAT * VOCAB, D)
  out = _sc_call(X, W_num, cat_flat, seq_table)
  return out.reshape(B, NSLOT, D)


# speedup vs baseline: 1.5585x; 1.5585x over previous
import jax
import jax.numpy as jnp
from jax import lax
from jax.experimental import pallas as pl
from jax.experimental.pallas import tpu as pltpu
from jax.experimental.pallas import tpu_sc as plsc

B = 4096
N_NUM = 13
N_CAT = 26
SEQ_LEN = 50
VOCAB = 100000
D = 32
NCOLS = N_NUM + N_CAT + SEQ_LEN
NSLOT = N_NUM + N_CAT + 1

NC, NS = 2, 16
NW = NC * NS
RPW = B // NW
C = 32
NCHUNK = RPW // C

CAT_ROWS = N_CAT * C
SEQ_ROWS = SEQ_LEN * C
NP_ROWS = (N_NUM + 1) * C
NP_W = 112


def _body(x_hbm, w_hbm, cat_hbm, seq_hbm, out_hbm,
          xv, seq_v, cat_v, np_v, gidx_seq, gidx_cat,
          sidx_cat, sidx_np, rcp_v, wv,
          sem_seq, sem_cat, sem_out):
  wid = lax.axis_index("s") * NC + lax.axis_index("c")
  lane = lax.iota(jnp.int32, 16)

  pltpu.sync_copy(w_hbm, wv)

  for ci in range(NCHUNK):
    gbase = wid * RPW + ci * C

    pltpu.sync_copy(x_hbm.at[pl.ds(gbase * NCOLS, C * NCOLS)], xv)

    rows89 = [(k * 16 + lane) * NCOLS for k in range(C // 16)]

    for j in range(N_CAT):
      for k in range(C // 16):
        v = plsc.load_gather(xv, [rows89[k] + (N_NUM + j)])
        gidx_cat[pl.ds(j * C + k * 16, 16)] = v + j * VOCAB
    cat_descs = []
    for r in range(6):
      cat_descs.append(pltpu.async_copy(
          cat_hbm.at[gidx_cat.at[pl.ds(r * 128, 128)]],
          cat_v.at[pl.ds(r * 128, 128)], sem_cat))
    cat_descs.append(pltpu.async_copy(
        cat_hbm.at[gidx_cat.at[pl.ds(768, 64)]],
        cat_v.at[pl.ds(768, 64)], sem_cat))

    cnt = [jnp.zeros((16,), jnp.float32) for _ in range(C // 16)]
    for t in range(SEQ_LEN):
      for k in range(C // 16):
        rows = k * 16 + lane
        v = plsc.load_gather(xv, [rows89[k] + (N_NUM + N_CAT + t)])
        gidx_seq[pl.ds(t * C + k * 16, 16)] = v
        cnt[k] = cnt[k] + jnp.where(v != 0, 1.0, 0.0)
    for k in range(C // 16):
      rcp_v[pl.ds(k * 16, 16)] = 1.0 / jnp.maximum(cnt[k], 1e-12)
    seq_descs = []
    for r in range(12):
      seq_descs.append(pltpu.async_copy(
          seq_hbm.at[gidx_seq.at[pl.ds(r * 128, 128)]],
          seq_v.at[pl.ds(r * 128, 128)], sem_seq))
    seq_descs.append(pltpu.async_copy(
        seq_hbm.at[gidx_seq.at[pl.ds(1536, 64)]],
        seq_v.at[pl.ds(1536, 64)], sem_seq))

    for j in range(N_CAT):
      for k in range(C // 16):
        orow = (gbase + k * 16 + lane) * NSLOT + N_NUM + j
        sidx_cat[j // 2, pl.ds((j % 2) * C + k * 16, 16)] = orow
    for i in range(N_NUM + 1):
      slot = i if i < N_NUM else NSLOT - 1
      for k in range(C // 16):
        bb = k * 16 + lane
        flat = bb * (N_NUM + 1) + i
        orow = (gbase + bb) * NSLOT + slot
        plsc.store_scatter(sidx_np, [flat // NP_W, flat % NP_W], orow)

    for dsc in seq_descs:
      dsc.wait()

    def row_body(b, carry):
      base = b * (N_NUM + 1)
      for i in range(N_NUM):
        xi = plsc.load_gather(
            xv, [jnp.full((16,), b * NCOLS + i, jnp.int32)]).astype(jnp.float32)
        np_v[base + i, pl.ds(0, 16)] = xi * wv[i, pl.ds(0, 16)]
        np_v[base + i, pl.ds(16, 16)] = xi * wv[i, pl.ds(16, 16)]
      acc0 = jnp.zeros((16,), jnp.float32)
      acc1 = jnp.zeros((16,), jnp.float32)
      for t in range(SEQ_LEN):
        acc0 = acc0 + seq_v[t * C + b, pl.ds(0, 16)]
        acc1 = acc1 + seq_v[t * C + b, pl.ds(16, 16)]
      rcp = plsc.load_gather(rcp_v, [jnp.full((16,), b, jnp.int32)])
      np_v[base + N_NUM, pl.ds(0, 16)] = acc0 * rcp
      np_v[base + N_NUM, pl.ds(16, 16)] = acc1 * rcp
      return carry

    lax.fori_loop(0, C, row_body, 0)

    out_descs = []
    for r in range(NP_ROWS // NP_W):
      out_descs.append(pltpu.async_copy(
          np_v.at[pl.ds(r * NP_W, NP_W)],
          out_hbm.at[sidx_np.at[r]], sem_out))

    for dsc in cat_descs:
      dsc.wait()
    for r in range(N_CAT // 2):
      out_descs.append(pltpu.async_copy(
          cat_v.at[pl.ds(r * 2 * C, 2 * C)],
          out_hbm.at[sidx_cat.at[r]], sem_out))

    for dsc in out_descs:
      dsc.wait()


_sc_call = pl.kernel(
    _body,
    out_type=jax.ShapeDtypeStruct((B * NSLOT, D), jnp.float32),
    mesh=plsc.VectorSubcoreMesh(core_axis_name="c", subcore_axis_name="s"),
    compiler_params=pltpu.CompilerParams(
        needs_layout_passes=False, use_tc_tiling_on_sc=False),
    scratch_types=[
        pltpu.VMEM((C * NCOLS,), jnp.int32),
        pltpu.VMEM((SEQ_ROWS, D), jnp.float32),
        pltpu.VMEM((CAT_ROWS, D), jnp.float32),
        pltpu.VMEM((NP_ROWS, D), jnp.float32),
        pltpu.VMEM((SEQ_ROWS,), jnp.int32),
        pltpu.VMEM((CAT_ROWS,), jnp.int32),
        pltpu.VMEM((N_CAT // 2, 2 * C), jnp.int32),
        pltpu.VMEM((NP_ROWS // NP_W, NP_W), jnp.int32),
        pltpu.VMEM((C,), jnp.float32),
        pltpu.VMEM((N_NUM, D), jnp.float32),
        pltpu.SemaphoreType.DMA,
        pltpu.SemaphoreType.DMA,
        pltpu.SemaphoreType.DMA,
    ],
)


@jax.jit
def kernel(X, W_num, cat_tables, seq_table):
  cat_flat = cat_tables.reshape(N_CAT * VOCAB, D)
  out = _sc_call(X.reshape(B * NCOLS), W_num, cat_flat, seq_table)
  return out.reshape(B, NSLOT, D)
